# Initial kernel scaffold; baseline (speedup 1.0000x reference)
import functools

import jax
import jax.numpy as jnp
from jax import lax
from jax.experimental import pallas as pl
from jax.experimental.pallas import tpu as pltpu
from jax.experimental.pallas import tpu_sc as plsc

USER = 5000
ITEM = 5000
N = USER + ITEM
LATDIM = 128
FEAT = 256
E = 320000
EPS = 0.1

NPAD = 10240
CH = 128
TCH = 157
EPT = TCH * CH
EPAD = 16 * EPT
HALF = LATDIM // 2


def _mlp_body(x_ref, w1_ref, b1_ref, w2_ref, b2_ref, o_ref):
    h = jnp.dot(x_ref[...], w1_ref[...], preferred_element_type=jnp.float32)
    h = jnp.maximum(h + b1_ref[...], 0.0)
    f = jnp.dot(h, w2_ref[...], preferred_element_type=jnp.float32)
    f = f + b2_ref[...]
    nrm = jnp.sqrt(jnp.sum(f * f, axis=1, keepdims=True))
    o_ref[...] = f / jnp.maximum(nrm, 1e-12)


def _mlp_norm(x, w1, b1, w2, b2):
    blk = 1000
    grid = (ITEM // blk,)
    return pl.pallas_call(
        _mlp_body,
        grid=grid,
        in_specs=[
            pl.BlockSpec((blk, FEAT), lambda i: (i, 0)),
            pl.BlockSpec((FEAT, LATDIM), lambda i: (0, 0)),
            pl.BlockSpec((1, LATDIM), lambda i: (0, 0)),
            pl.BlockSpec((LATDIM, LATDIM), lambda i: (0, 0)),
            pl.BlockSpec((1, LATDIM), lambda i: (0, 0)),
        ],
        out_specs=pl.BlockSpec((blk, LATDIM), lambda i: (i, 0)),
        out_shape=jax.ShapeDtypeStruct((ITEM, LATDIM), jnp.float32),
    )(x, w1, b1.reshape(1, LATDIM), w2, b2.reshape(1, LATDIM))


def _sc_body(xini, src2, dst2, wf, u00, u10, u01, u11,
             out_main, out_v1, out_v2, sp_e1, sp_b, sp_c,
             srcs_v, dsts_v, w_v, rows_v, t1, t2, t3, t4, acc, sem):
    c = lax.axis_index("c")
    s = lax.axis_index("s")
    rowbase = c * NPAD

    pltpu.sync_copy(src2.at[pl.ds(s * TCH, TCH)], srcs_v)
    pltpu.sync_copy(dst2.at[pl.ds(s * TCH, TCH)], dsts_v)
    pltpu.sync_copy(wf.at[pl.ds(s * EPT, EPT)], w_v)

    def _off(t, _):
        for g in range(CH // 16):
            sl = (t, pl.ds(g * 16, 16))
            srcs_v[sl] = srcs_v[sl] + rowbase
        return 0
    lax.fori_loop(0, TCH, _off, 0)

    def _zero_acc():
        def _z(r, _):
            for g in range(HALF // 16):
                rows_v[r, pl.ds(g * 16, 16)] = jnp.zeros((16,), jnp.float32)
            return 0
        lax.fori_loop(0, CH, _z, 0)
        for k in range(NPAD // (16 * CH)):
            pltpu.sync_copy(rows_v, acc.at[pl.ds((s * 5 + k) * CH, CH)])

    def _spmm_pass(xref):
        _zero_acc()
        plsc.subcore_barrier()

        def _chunk(i, _):
            pltpu.async_copy(xref.at[srcs_v.at[i]], rows_v, sem).wait()
            base_fl = i * CH

            def _edge(e, _2):
                wb = plsc.load_gather(
                    w_v, [jnp.full((16,), base_fl + e, jnp.int32)])
                for g in range(HALF // 16):
                    sl = (e, pl.ds(g * 16, 16))
                    rows_v[sl] = rows_v[sl] * wb
                return 0
            lax.fori_loop(0, CH, _edge, 0)
            pltpu.sync_copy(rows_v, acc.at[dsts_v.at[i]], add=True)
            return 0
        lax.fori_loop(0, TCH, _chunk, 0)
        plsc.subcore_barrier()

    _spmm_pass(xini)
    for k in range(5):
        r0 = (s * 5 + k) * CH
        ro = rowbase + r0
        pltpu.sync_copy(acc.at[pl.ds(r0, CH)], rows_v)
        pltpu.sync_copy(u00.at[pl.ds(ro, CH)], t1)
        pltpu.sync_copy(u10.at[pl.ds(ro, CH)], t2)

        def _ew(r, _):
            for g in range(HALF // 16):
                sl = (r, pl.ds(g * 16, 16))
                e1 = rows_v[sl]
                sg = jnp.sign(e1) * EPS
                t3[sl] = e1 + t1[sl] * sg
                t4[sl] = e1 + t2[sl] * sg
            return 0
        lax.fori_loop(0, CH, _ew, 0)
        pltpu.sync_copy(rows_v, sp_e1.at[pl.ds(ro, CH)])
        pltpu.sync_copy(t3, sp_b.at[pl.ds(ro, CH)])
        pltpu.sync_copy(t4, sp_c.at[pl.ds(ro, CH)])
    plsc.subcore_barrier()

    for xref, outref, unoise in ((sp_e1, out_main, None),
                                 (sp_b, out_v1, u01),
                                 (sp_c, out_v2, u11)):
        _spmm_pass(xref)
        for k in range(5):
            r0 = (s * 5 + k) * CH
            ro = rowbase + r0
            pltpu.sync_copy(acc.at[pl.ds(r0, CH)], rows_v)
            pltpu.sync_copy(xini.at[pl.ds(ro, CH)], t1)
            pltpu.sync_copy(xref.at[pl.ds(ro, CH)], t2)
            if unoise is not None:
                pltpu.sync_copy(unoise.at[pl.ds(ro, CH)], t3)

            def _fin(r, _):
                for g in range(HALF // 16):
                    sl = (r, pl.ds(g * 16, 16))
                    s2 = rows_v[sl]
                    o = t1[sl] + t2[sl] + s2
                    if unoise is not None:
                        o = o + t3[sl] * (jnp.sign(s2) * EPS)
                    t4[sl] = o
                return 0
            lax.fori_loop(0, CH, _fin, 0)
            pltpu.sync_copy(t4, outref.at[pl.ds(ro, CH)])
        plsc.subcore_barrier()


_sc_gcn = pl.kernel(
    _sc_body,
    out_type=[jax.ShapeDtypeStruct((2 * NPAD, HALF), jnp.float32)] * 6,
    mesh=plsc.VectorSubcoreMesh(core_axis_name="c", subcore_axis_name="s"),
    scratch_types=[
        pltpu.VMEM((TCH, CH), jnp.int32),
        pltpu.VMEM((TCH, CH), jnp.int32),
        pltpu.VMEM((EPT,), jnp.float32),
        pltpu.VMEM((CH, HALF), jnp.float32),
        pltpu.VMEM((CH, HALF), jnp.float32),
        pltpu.VMEM((CH, HALF), jnp.float32),
        pltpu.VMEM((CH, HALF), jnp.float32),
        pltpu.VMEM((CH, HALF), jnp.float32),
        pltpu.VMEM_SHARED((NPAD, HALF), jnp.float32),
        pltpu.SemaphoreType.DMA,
    ],
)


def _stack_halves(a):
    p = jnp.pad(a, ((0, NPAD - N), (0, 0)))
    return jnp.concatenate([p[:, :HALF], p[:, HALF:]], axis=0)


def _unstack_halves(o):
    return jnp.concatenate([o[:N], o[NPAD:NPAD + N]], axis=1)


def kernel(edge_index, edge_weight, item_feats_trn, uEmbeds, W1, b1, W2, b2):
    if_n = _mlp_norm(item_feats_trn, W1, b1, W2, b2)
    ini = jnp.concatenate([uEmbeds, if_n], axis=0)

    pkey = jax.random.key(1234)
    us = []
    for i in (0, 1, 100, 101):
        u = jax.random.uniform(jax.random.fold_in(pkey, i), (N, LATDIM),
                               jnp.float32)
        nrm = jnp.linalg.norm(u, axis=1, keepdims=True)
        us.append(u / jnp.maximum(nrm, 1e-12))
    u00s, u01s, u10s, u11s = (_stack_halves(u) for u in us)

    src = edge_index[0]
    dst = edge_index[1]
    padn = EPAD - E
    padi = (jnp.arange(padn, dtype=jnp.int32) * 97) % N
    src2 = jnp.concatenate([src, padi]).reshape(16 * TCH, CH)
    dst2 = jnp.concatenate([dst, padi]).reshape(16 * TCH, CH)
    wf = jnp.concatenate([edge_weight, jnp.zeros((padn,), jnp.float32)])

    om, o1, o2, _, _, _ = _sc_gcn(_stack_halves(ini), src2, dst2, wf,
                                  u00s, u10s, u01s, u11s)
    main = _unstack_halves(om)
    v1 = _unstack_halves(o1)
    v2 = _unstack_halves(o2)
    return (main[:USER], if_n, v1[:USER], v1[USER:], v2[:USER], v2[USER:])


# speedup vs baseline: 5.2736x; 5.2736x over previous
import jax
import jax.numpy as jnp
from jax import lax
from jax.experimental import pallas as pl
from jax.experimental.pallas import tpu as pltpu
from jax.experimental.pallas import tpu_sc as plsc

USER = 5000
ITEM = 5000
N = USER + ITEM
LATDIM = 128
FEAT = 256
E = 320000
EPS = 0.1

NPAD = 10240
CH = 128
TCH = 80
EPT = TCH * CH
EPAD = 32 * EPT
RPT = NPAD // 16
KR = RPT // CH


def _mlp_body(x_ref, w1_ref, b1_ref, w2_ref, b2_ref, o_ref):
    h = jnp.dot(x_ref[...], w1_ref[...], preferred_element_type=jnp.float32)
    h = jnp.maximum(h + b1_ref[...], 0.0)
    f = jnp.dot(h, w2_ref[...], preferred_element_type=jnp.float32)
    f = f + b2_ref[...]
    nrm = jnp.sqrt(jnp.sum(f * f, axis=1, keepdims=True))
    o_ref[...] = f / jnp.maximum(nrm, 1e-12)


def _mlp_norm(x, w1, b1, w2, b2):
    blk = 1000
    return pl.pallas_call(
        _mlp_body,
        grid=(ITEM // blk,),
        in_specs=[
            pl.BlockSpec((blk, FEAT), lambda i: (i, 0)),
            pl.BlockSpec((FEAT, LATDIM), lambda i: (0, 0)),
            pl.BlockSpec((1, LATDIM), lambda i: (0, 0)),
            pl.BlockSpec((LATDIM, LATDIM), lambda i: (0, 0)),
            pl.BlockSpec((1, LATDIM), lambda i: (0, 0)),
        ],
        out_specs=pl.BlockSpec((blk, LATDIM), lambda i: (i, 0)),
        out_shape=jax.ShapeDtypeStruct((ITEM, LATDIM), jnp.float32),
    )(x, w1, b1.reshape(1, LATDIM), w2, b2.reshape(1, LATDIM))


def _make_sc_body(nviews):
    def body(*refs):
        xs = refs[:nviews]
        src2, dst2, wf = refs[nviews:nviews + 3]
        out = refs[nviews + 3]
        srcs_v, dsts_v, w_v, rows_v, acc, sem = refs[nviews + 4:]
        c = lax.axis_index("c")
        s = lax.axis_index("s")
        wid = c * 16 + s

        if True:
            pltpu.sync_copy(src2.at[pl.ds(wid * TCH, TCH)], srcs_v)
            pltpu.sync_copy(dst2.at[pl.ds(wid * TCH, TCH)], dsts_v)
            pltpu.sync_copy(wf.at[pl.ds(wid * EPT, EPT)], w_v)

            for v in range(nviews):
                def _z(r, _):
                    for g in range(LATDIM // 16):
                        rows_v[r, pl.ds(g * 16, 16)] = jnp.zeros(
                            (16,), jnp.float32)
                    return 0
                lax.fori_loop(0, CH, _z, 0)
                for k in range(KR):
                    pltpu.sync_copy(
                        rows_v, acc.at[pl.ds((s * KR + k) * CH, CH)])
                plsc.subcore_barrier()

                xref = xs[v]

                def _chunk(i, _):
                    pltpu.async_copy(
                        xref.at[srcs_v.at[i]], rows_v, sem).wait()
                    base_fl = i * CH

                    def _edge(e, _2):
                        wb = plsc.load_gather(
                            w_v, [jnp.full((16,), base_fl + e, jnp.int32)])
                        for g in range(LATDIM // 16):
                            sl = (e, pl.ds(g * 16, 16))
                            rows_v[sl] = rows_v[sl] * wb
                        return 0
                    lax.fori_loop(0, CH, _edge, 0)
                    pltpu.sync_copy(rows_v, acc.at[dsts_v.at[i]], add=True)
                    return 0
                lax.fori_loop(0, TCH, _chunk, 0)
                plsc.subcore_barrier()

                for k in range(KR):
                    r0 = (s * KR + k) * CH
                    pltpu.sync_copy(acc.at[pl.ds(r0, CH)], rows_v)
                    pltpu.sync_copy(
                        rows_v, out.at[2 * v + c, pl.ds(r0, CH)])

    return body


def _sc_spmm(xs, src2, dst2, wf):
    nviews = len(xs)
    f = pl.kernel(
        _make_sc_body(nviews),
        out_type=jax.ShapeDtypeStruct((2 * nviews, NPAD, LATDIM),
                                      jnp.float32),
        mesh=plsc.VectorSubcoreMesh(core_axis_name="c", subcore_axis_name="s"),
        compiler_params=pltpu.CompilerParams(needs_layout_passes=False),
        scratch_types=[
            pltpu.VMEM((TCH, CH), jnp.int32),
            pltpu.VMEM((TCH, CH), jnp.int32),
            pltpu.VMEM((EPT,), jnp.float32),
            pltpu.VMEM((CH, LATDIM), jnp.float32),
            pltpu.VMEM_SHARED((NPAD, LATDIM), jnp.float32),
            pltpu.SemaphoreType.DMA,
        ],
    )
    return f(*xs, src2, dst2, wf)


_EW_BLK = 1280


def _ew_spec():
    return pl.BlockSpec((_EW_BLK, LATDIM), lambda i: (i, 0))


def _perturb_body(p0_ref, p1_ref, ua_ref, ub_ref, e1_ref, b_ref, c_ref):
    e1 = p0_ref[...] + p1_ref[...]
    sg = jnp.sign(e1) * EPS
    e1_ref[...] = e1
    b_ref[...] = e1 + ua_ref[...] * sg
    c_ref[...] = e1 + ub_ref[...] * sg


def _perturb_tc(p0, p1, ua, ub):
    return pl.pallas_call(
        _perturb_body,
        grid=(NPAD // _EW_BLK,),
        in_specs=[_ew_spec()] * 4,
        out_specs=[_ew_spec()] * 3,
        out_shape=[jax.ShapeDtypeStruct((NPAD, LATDIM), jnp.float32)] * 3,
    )(p0, p1, ua, ub)


def _final_body(ini_ref, e1_ref, b_ref, c_ref,
                qa0_ref, qa1_ref, qb0_ref, qb1_ref, qc0_ref, qc1_ref,
                ua_ref, ub_ref, m_ref, v1_ref, v2_ref):
    ini = ini_ref[...]
    s2a = qa0_ref[...] + qa1_ref[...]
    m_ref[...] = ini + e1_ref[...] + s2a
    s2b = qb0_ref[...] + qb1_ref[...]
    v1_ref[...] = ini + b_ref[...] + s2b + ua_ref[...] * (jnp.sign(s2b) * EPS)
    s2c = qc0_ref[...] + qc1_ref[...]
    v2_ref[...] = ini + c_ref[...] + s2c + ub_ref[...] * (jnp.sign(s2c) * EPS)


def _final_tc(ini, e1, b, c, q, ua, ub):
    return pl.pallas_call(
        _final_body,
        grid=(NPAD // _EW_BLK,),
        in_specs=[_ew_spec()] * 12,
        out_specs=[_ew_spec()] * 3,
        out_shape=[jax.ShapeDtypeStruct((NPAD, LATDIM), jnp.float32)] * 3,
    )(ini, e1, b, c, q[0], q[1], q[2], q[3], q[4], q[5], ua, ub)


def _padrows(a):
    return jnp.pad(a, ((0, NPAD - N), (0, 0)))


def kernel(edge_index, edge_weight, item_feats_trn, uEmbeds, W1, b1, W2, b2):
    if_n = _mlp_norm(item_feats_trn, W1, b1, W2, b2)
    ini = _padrows(jnp.concatenate([uEmbeds, if_n], axis=0))

    pkey = jax.random.key(1234)
    us = []
    for i in (0, 1, 100, 101):
        u = jax.random.uniform(jax.random.fold_in(pkey, i), (N, LATDIM),
                               jnp.float32)
        nrm = jnp.linalg.norm(u, axis=1, keepdims=True)
        us.append(_padrows(u / jnp.maximum(nrm, 1e-12)))
    u00, u01, u10, u11 = us

    src = edge_index[0]
    dst = edge_index[1]
    padn = EPAD - E
    padi = (jnp.arange(padn, dtype=jnp.int32) * 97) % N
    src2 = jnp.concatenate([src, padi]).reshape(EPAD // CH, CH)
    dst2 = jnp.concatenate([dst, padi]).reshape(EPAD // CH, CH)
    wf = jnp.concatenate([edge_weight, jnp.zeros((padn,), jnp.float32)])

    p = _sc_spmm([ini], src2, dst2, wf)
    e1, bb, cc = _perturb_tc(p[0], p[1], u00, u10)
    q = _sc_spmm([e1, bb, cc], src2, dst2, wf)
    main, v1, v2 = _final_tc(ini, e1, bb, cc, q, u01, u11)

    return (main[:USER], if_n, v1[:USER], v1[USER:N],
            v2[:USER], v2[USER:N])
